# baseline (speedup 1.0000x reference)
import dataclasses
import functools

import jax
import jax.numpy as jnp
from jax import lax
from jax.experimental import pallas as pl
from jax.experimental.pallas import tpu as pltpu
from jax.experimental.pallas import tpu_sc as plsc

S = 2048
H = 768
F = 2048
E = 64
TOPK = 2
A = S * TOPK
TROW = 128
OVF = 31
MAIN_ROWS = E * TROW
MAX_ROWS = MAIN_ROWS + OVF * TROW

_NC = 2
_NS = 16
_NW = _NC * _NS
_SBLK = 256
_NBLK = S // _SBLK


def _expert_mlp(xb, w1_ref, w3_ref, w2_ref):
    a = lax.dot_general(xb, w1_ref[0].astype(jnp.bfloat16),
                        (((1,), (1,)), ((), ())),
                        preferred_element_type=jnp.float32)
    b = lax.dot_general(xb, w3_ref[0].astype(jnp.bfloat16),
                        (((1,), (1,)), ((), ())),
                        preferred_element_type=jnp.float32)
    h = (a * jax.nn.sigmoid(a)) * b
    return lax.dot_general(h.astype(jnp.bfloat16),
                           w2_ref[0].astype(jnp.bfloat16),
                           (((1,), (1,)), ((), ())),
                           preferred_element_type=jnp.float32)


def _onehot_rows(pr_ref, row_lo):
    rowidx = row_lo + lax.broadcasted_iota(jnp.int32, (S, TROW), 1)
    return ((pr_ref[:, 0:1] == rowidx) |
            (pr_ref[:, 1:2] == rowidx)).astype(jnp.bfloat16)


def _mega_body(x_ref, wg_ref, w1_ref, w3_ref, w2_ref,
               lg_ref, wt_ref, prow_ref, ote_ref, novf_ref, y_ref,
               x16_s, sel_s, rank_s):
    i = pl.program_id(0)

    @pl.when(i == 0)
    def _():
        x16_s[...] = x_ref[...].astype(jnp.bfloat16)
        cnt = jnp.zeros((1, E), jnp.float32)
        rr = lax.broadcasted_iota(jnp.int32, (_SBLK, _SBLK), 0)
        cc = lax.broadcasted_iota(jnp.int32, (_SBLK, _SBLK), 1)
        ltri = (rr > cc).astype(jnp.bfloat16)
        for b in range(_NBLK):
            xb = x_ref[pl.ds(b * _SBLK, _SBLK), :]
            lg = lax.dot_general(xb, wg_ref[...], (((1,), (1,)), ((), ())),
                                 preferred_element_type=jnp.float32)
            lg_ref[pl.ds(b * _SBLK, _SBLK), :] = lg
            iota = lax.broadcasted_iota(jnp.int32, lg.shape, 1)
            m1 = jnp.max(lg, axis=1, keepdims=True)
            e1 = jnp.min(jnp.where(lg == m1, iota, E), axis=1, keepdims=True)
            lg2 = jnp.where(iota == e1, -jnp.inf, lg)
            m2 = jnp.max(lg2, axis=1, keepdims=True)
            e2 = jnp.min(jnp.where(lg2 == m2, iota, E), axis=1, keepdims=True)
            w1 = 1.0 / (1.0 + jnp.exp(m2 - m1))
            wt_ref[pl.ds(b * _SBLK, _SBLK), :] = jnp.concatenate(
                [w1, 1.0 - w1], axis=1)
            sel_s[pl.ds(b * _SBLK, _SBLK), :] = jnp.concatenate(
                [e1, e2], axis=1)
            oh0 = (iota == e1).astype(jnp.bfloat16)
            oh1 = (iota == e2).astype(jnp.bfloat16)
            p0 = lax.dot_general(ltri, oh0, (((1,), (0,)), ((), ())),
                                 preferred_element_type=jnp.float32)
            p1 = lax.dot_general(ltri, oh1, (((1,), (0,)), ((), ())),
                                 preferred_element_type=jnp.float32)
            oh0f = oh0.astype(jnp.float32)
            oh1f = oh1.astype(jnp.float32)
            s0 = jnp.sum(oh0f, axis=0, keepdims=True)
            rank0 = jnp.sum(oh0f * (p0 + cnt), axis=1, keepdims=True)
            rank1 = jnp.sum(oh1f * (p1 + cnt + s0), axis=1, keepdims=True)
            rank_s[pl.ds(b * _SBLK, _SBLK), :] = jnp.concatenate(
                [rank0, rank1], axis=1)
            cnt = cnt + s0 + jnp.sum(oh1f, axis=0, keepdims=True)

        ovf_t = jnp.maximum(
            jnp.floor((cnt + (TROW - 1)) * (1.0 / TROW)) - 1.0, 0.0)
        iot = lax.broadcasted_iota(jnp.int32, (E, E), 0)
        iot2 = lax.broadcasted_iota(jnp.int32, (E, E), 1)
        ut = (iot <= iot2).astype(jnp.bfloat16)
        cum_ovf = lax.dot_general(ovf_t.astype(jnp.bfloat16), ut,
                                  (((1,), (0,)), ((), ())),
                                  preferred_element_type=jnp.float32)
        ovf_start = MAIN_ROWS + (cum_ovf - ovf_t) * TROW
        novf_ref[...] = cum_ovf[:, E - 1:].astype(jnp.int32)
        tt = lax.broadcasted_iota(jnp.int32, (OVF, E), 0)
        cum_i = cum_ovf.astype(jnp.int32)
        ote = jnp.sum((jnp.broadcast_to(cum_i, (OVF, E)) <= tt)
                      .astype(jnp.int32), axis=1, keepdims=True)
        ote_ref[...] = jnp.minimum(ote, E - 1)
        sel = sel_s[...]
        rank = rank_s[...]
        iota_e = lax.broadcasted_iota(jnp.int32, (S, E), 1)
        ovb = jnp.broadcast_to(ovf_start, (S, E))
        ov0 = jnp.sum(jnp.where(iota_e == sel[:, 0:1], ovb, 0.0),
                      axis=1, keepdims=True)
        ov1 = jnp.sum(jnp.where(iota_e == sel[:, 1:2], ovb, 0.0),
                      axis=1, keepdims=True)
        self_rows = sel.astype(jnp.float32) * TROW + rank
        ovf_rows = (jnp.concatenate([ov0, ov1], axis=1) + rank - TROW)
        prow_ref[...] = jnp.where(rank < TROW, self_rows,
                                  ovf_rows).astype(jnp.int32)

    @pl.when(i > 0)
    def _():
        oh = _onehot_rows(prow_ref, (i - 1) * TROW)
        xb = lax.dot_general(oh, x16_s[...], (((0,), (0,)), ((), ())),
                             preferred_element_type=jnp.float32
                             ).astype(jnp.bfloat16)
        y_ref[...] = _expert_mlp(xb, w1_ref, w3_ref, w2_ref)


def _mega(x, wg, w1, w3, w2):
    widx = lambda i: (jnp.maximum(i - 1, 0), 0, 0)
    return pl.pallas_call(
        _mega_body,
        grid=(1 + E,),
        in_specs=[
            pl.BlockSpec((S, H), lambda i: (0, 0)),
            pl.BlockSpec((E, H), lambda i: (0, 0)),
            pl.BlockSpec((1, F, H), widx),
            pl.BlockSpec((1, F, H), widx),
            pl.BlockSpec((1, H, F), widx),
        ],
        out_specs=[
            pl.BlockSpec((S, E), lambda i: (0, 0)),
            pl.BlockSpec((S, TOPK), lambda i: (0, 0)),
            pl.BlockSpec((S, TOPK), lambda i: (0, 0)),
            pl.BlockSpec((OVF, 1), lambda i: (0, 0)),
            pl.BlockSpec((1, 1), lambda i: (0, 0)),
            pl.BlockSpec((TROW, H), lambda i: (jnp.maximum(i - 1, 0), 0)),
        ],
        out_shape=[
            jax.ShapeDtypeStruct((S, E), jnp.float32),
            jax.ShapeDtypeStruct((S, TOPK), jnp.float32),
            jax.ShapeDtypeStruct((S, TOPK), jnp.int32),
            jax.ShapeDtypeStruct((OVF, 1), jnp.int32),
            jax.ShapeDtypeStruct((1, 1), jnp.int32),
            jax.ShapeDtypeStruct((MAX_ROWS, H), jnp.float32),
        ],
        scratch_shapes=[
            pltpu.VMEM((S, H), jnp.bfloat16),
            pltpu.VMEM((S, TOPK), jnp.int32),
            pltpu.VMEM((S, TOPK), jnp.float32),
        ],
        compiler_params=pltpu.CompilerParams(
            dimension_semantics=("arbitrary",)),
    )(x, wg, w1, w3, w2)


def _ovf_body(ote_ref, novf_ref, x_ref, pr_ref, w1_ref, w3_ref, w2_ref,
              y_in_ref, y_ref):
    j = pl.program_id(0)
    del y_in_ref

    @pl.when(j < novf_ref[0])
    def _():
        oh = _onehot_rows(pr_ref, MAIN_ROWS + j * TROW)
        xb = lax.dot_general(oh, x_ref[...].astype(jnp.bfloat16),
                             (((0,), (0,)), ((), ())),
                             preferred_element_type=jnp.float32
                             ).astype(jnp.bfloat16)
        y_ref[...] = _expert_mlp(xb, w1_ref, w3_ref, w2_ref)


def _moe_overflow(x, prow2, w1, w3, w2, ote, novf, y):
    def we_idx(j, ote_, novf_):
        safe = jnp.minimum(j, jnp.maximum(novf_[0] - 1, 0))
        return (ote_[safe], 0, 0)

    grid_spec = pltpu.PrefetchScalarGridSpec(
        num_scalar_prefetch=2,
        grid=(OVF,),
        in_specs=[
            pl.BlockSpec((S, H), lambda j, ote_, novf_: (0, 0)),
            pl.BlockSpec((S, TOPK), lambda j, ote_, novf_: (0, 0)),
            pl.BlockSpec((1, F, H), we_idx),
            pl.BlockSpec((1, F, H), we_idx),
            pl.BlockSpec((1, H, F), we_idx),
            pl.BlockSpec((TROW, H),
                         lambda j, ote_, novf_: (E + j, 0)),
        ],
        out_specs=pl.BlockSpec((TROW, H),
                               lambda j, ote_, novf_: (E + j, 0)),
    )
    return pl.pallas_call(
        _ovf_body,
        grid_spec=grid_spec,
        out_shape=jax.ShapeDtypeStruct((MAX_ROWS, H), jnp.float32),
        input_output_aliases={7: 0},
        compiler_params=pltpu.CompilerParams(
            dimension_semantics=("arbitrary",)),
    )(ote, novf, x, prow2, w1, w3, w2, y)


_TPW = S // _NW


def _sc_cparams():
    cp = pltpu.CompilerParams()
    if "needs_layout_passes" in pltpu.CompilerParams.__dataclass_fields__:
        cp = dataclasses.replace(cp, needs_layout_passes=False)
    return cp


def _sc_combine(y, pos0, pos1, w_flat):
    @functools.partial(
        pl.kernel,
        out_type=jax.ShapeDtypeStruct((S, H), jnp.float32),
        mesh=plsc.VectorSubcoreMesh(core_axis_name="c", subcore_axis_name="s"),
        compiler_params=_sc_cparams(),
        scratch_types=[
            pltpu.VMEM((_TPW,), jnp.int32),
            pltpu.VMEM((_TPW,), jnp.int32),
            pltpu.VMEM((2 * _TPW,), jnp.float32),
            pltpu.VMEM((_TPW, H), jnp.float32),
            pltpu.VMEM((_TPW, H), jnp.float32),
            pltpu.SemaphoreType.DMA,
        ],
    )
    def k(y_hbm, p0_hbm, p1_hbm, w_hbm, out_hbm, i0, i1, wv, b0, b1, sem):
        wid = lax.axis_index("s") * _NC + lax.axis_index("c")
        base = wid * _TPW
        pltpu.sync_copy(p0_hbm.at[pl.ds(base, _TPW)], i0)
        pltpu.sync_copy(p1_hbm.at[pl.ds(base, _TPW)], i1)
        pltpu.sync_copy(w_hbm.at[pl.ds(2 * base, 2 * _TPW)], wv)
        c0 = pltpu.async_copy(y_hbm.at[i0], b0, sem)
        c1 = pltpu.async_copy(y_hbm.at[i1], b1, sem)
        c0.wait()
        c1.wait()

        @pl.loop(0, _TPW)
        def _(r):
            w0 = plsc.load_gather(wv, [jnp.full((16,), 2 * r, jnp.int32)])
            w1 = plsc.load_gather(wv, [jnp.full((16,), 2 * r + 1, jnp.int32)])

            @pl.loop(0, H, step=16)
            def _(c):
                b0[r, pl.ds(c, 16)] = (b0[r, pl.ds(c, 16)] * w0 +
                                       b1[r, pl.ds(c, 16)] * w1)

        pltpu.sync_copy(b0, out_hbm.at[pl.ds(base, _TPW)])

    return k(y, pos0, pos1, w_flat)


def kernel(hidden_states, Wg, W1, W3, W2):
    x = hidden_states.reshape(S, H)
    logits, wts, prow2, ote, novf, y = _mega(x, Wg, W1, W3, W2)
    y = _moe_overflow(x, prow2, W1, W3, W2,
                      ote.reshape(OVF), novf.reshape(1), y)
    final = _sc_combine(y, prow2[:, 0], prow2[:, 1], wts.reshape(A))
    return final.reshape(hidden_states.shape), logits


# speedup vs baseline: 1.0205x; 1.0205x over previous
import dataclasses
import functools

import jax
import jax.numpy as jnp
from jax import lax
from jax.experimental import pallas as pl
from jax.experimental.pallas import tpu as pltpu
from jax.experimental.pallas import tpu_sc as plsc

S = 2048
H = 768
F = 2048
E = 64
TOPK = 2
A = S * TOPK
TROW = 128
MAX_TILES = 96
MAX_ROWS = MAX_TILES * TROW

_NC = 2
_NS = 16
_NW = _NC * _NS
_SBLK = 256


_NBLK = S // _SBLK


def _router_body(x_ref, wg_ref, lg_ref, wt_ref, prow_ref, te_ref, nr_ref,
                 cnt_ref, sel_s, rank_s):
    i = pl.program_id(0)

    @pl.when(i == 0)
    def _():
        cnt_ref[...] = jnp.zeros((1, E), jnp.float32)

    @pl.when(i < _NBLK)
    def _():
        lg = lax.dot_general(x_ref[...], wg_ref[...], (((1,), (1,)), ((), ())),
                             preferred_element_type=jnp.float32)
        lg_ref[...] = lg
        iota = lax.broadcasted_iota(jnp.int32, lg.shape, 1)
        m1 = jnp.max(lg, axis=1, keepdims=True)
        e1 = jnp.min(jnp.where(lg == m1, iota, E), axis=1, keepdims=True)
        lg2 = jnp.where(iota == e1, -jnp.inf, lg)
        m2 = jnp.max(lg2, axis=1, keepdims=True)
        e2 = jnp.min(jnp.where(lg2 == m2, iota, E), axis=1, keepdims=True)
        w1 = 1.0 / (1.0 + jnp.exp(m2 - m1))
        wt_ref[...] = jnp.concatenate([w1, 1.0 - w1], axis=1)
        sel_s[pl.ds(i * _SBLK, _SBLK), :] = jnp.concatenate([e1, e2], axis=1)

        oh0 = (iota == e1).astype(jnp.bfloat16)
        oh1 = (iota == e2).astype(jnp.bfloat16)
        rr = lax.broadcasted_iota(jnp.int32, (_SBLK, _SBLK), 0)
        cc = lax.broadcasted_iota(jnp.int32, (_SBLK, _SBLK), 1)
        ltri = (rr > cc).astype(jnp.bfloat16)
        p0 = lax.dot_general(ltri, oh0, (((1,), (0,)), ((), ())),
                             preferred_element_type=jnp.float32)
        p1 = lax.dot_general(ltri, oh1, (((1,), (0,)), ((), ())),
                             preferred_element_type=jnp.float32)
        oh0f = oh0.astype(jnp.float32)
        oh1f = oh1.astype(jnp.float32)
        s0 = jnp.sum(oh0f, axis=0, keepdims=True)
        cnt = cnt_ref[...]
        rank0 = jnp.sum(oh0f * (p0 + cnt), axis=1, keepdims=True)
        rank1 = jnp.sum(oh1f * (p1 + cnt + s0), axis=1, keepdims=True)
        rank_s[pl.ds(i * _SBLK, _SBLK), :] = jnp.concatenate(
            [rank0, rank1], axis=1)
        cnt_ref[...] = cnt + s0 + jnp.sum(oh1f, axis=0, keepdims=True)

    @pl.when(i == _NBLK)
    def _():
        cnt = cnt_ref[...]
        n_tiles = jnp.floor((cnt + (TROW - 1)) * (1.0 / TROW))
        iot = lax.broadcasted_iota(jnp.int32, (E, E), 0)
        iot2 = lax.broadcasted_iota(jnp.int32, (E, E), 1)
        ut = (iot <= iot2).astype(jnp.bfloat16)
        cum_tiles = lax.dot_general(n_tiles.astype(jnp.bfloat16), ut,
                                    (((1,), (0,)), ((), ())),
                                    preferred_element_type=jnp.float32)
        row_base = (cum_tiles - n_tiles) * TROW
        nr_ref[...] = cum_tiles[:, E - 1:].astype(jnp.int32)
        tt = lax.broadcasted_iota(jnp.int32, (MAX_TILES, E), 0)
        cum_i = cum_tiles.astype(jnp.int32)
        te = jnp.sum((jnp.broadcast_to(cum_i, (MAX_TILES, E)) <= tt)
                     .astype(jnp.int32), axis=1, keepdims=True)
        te_ref[...] = jnp.minimum(te, E - 1)
        sel = sel_s[...]
        iota_e0 = lax.broadcasted_iota(jnp.int32, (S, E), 1)
        rb = jnp.broadcast_to(row_base, (S, E))
        rb0 = jnp.sum(jnp.where(iota_e0 == sel[:, 0:1], rb, 0.0),
                      axis=1, keepdims=True)
        rb1 = jnp.sum(jnp.where(iota_e0 == sel[:, 1:2], rb, 0.0),
                      axis=1, keepdims=True)
        prow_ref[...] = (jnp.concatenate([rb0, rb1], axis=1) +
                         rank_s[...]).astype(jnp.int32)


def _router(x, wg):
    return pl.pallas_call(
        _router_body,
        grid=(_NBLK + 1,),
        in_specs=[
            pl.BlockSpec((_SBLK, H), lambda i: (jnp.minimum(i, _NBLK - 1), 0)),
            pl.BlockSpec((E, H), lambda i: (0, 0)),
        ],
        out_specs=[
            pl.BlockSpec((_SBLK, E), lambda i: (jnp.minimum(i, _NBLK - 1), 0)),
            pl.BlockSpec((_SBLK, TOPK),
                         lambda i: (jnp.minimum(i, _NBLK - 1), 0)),
            pl.BlockSpec((S, TOPK), lambda i: (0, 0)),
            pl.BlockSpec((MAX_TILES, 1), lambda i: (0, 0)),
            pl.BlockSpec((1, 1), lambda i: (0, 0)),
        ],
        out_shape=[
            jax.ShapeDtypeStruct((S, E), jnp.float32),
            jax.ShapeDtypeStruct((S, TOPK), jnp.float32),
            jax.ShapeDtypeStruct((S, TOPK), jnp.int32),
            jax.ShapeDtypeStruct((MAX_TILES, 1), jnp.int32),
            jax.ShapeDtypeStruct((1, 1), jnp.int32),
        ],
        scratch_shapes=[
            pltpu.VMEM((1, E), jnp.float32),
            pltpu.VMEM((S, TOPK), jnp.int32),
            pltpu.VMEM((S, TOPK), jnp.float32),
        ],
    )(x, wg)


def _mlp_body(te_ref, nr_ref, x_ref, pr_ref, w1_ref, w3_ref, w2_ref, y_ref):
    i = pl.program_id(0)

    @pl.when(i < nr_ref[0])
    def _():
        rowidx = i * TROW + lax.broadcasted_iota(jnp.int32, (S, TROW), 1)
        oh = ((pr_ref[:, 0:1] == rowidx) |
              (pr_ref[:, 1:2] == rowidx)).astype(jnp.bfloat16)
        xb = lax.dot_general(oh, x_ref[...], (((0,), (0,)), ((), ())),
                             preferred_element_type=jnp.float32
                             ).astype(jnp.bfloat16)
        a = lax.dot_general(xb, w1_ref[0].astype(jnp.bfloat16),
                            (((1,), (1,)), ((), ())),
                            preferred_element_type=jnp.float32)
        b = lax.dot_general(xb, w3_ref[0].astype(jnp.bfloat16),
                            (((1,), (1,)), ((), ())),
                            preferred_element_type=jnp.float32)
        h = (a * jax.nn.sigmoid(a)) * b
        y_ref[...] = lax.dot_general(h.astype(jnp.bfloat16),
                                     w2_ref[0].astype(jnp.bfloat16),
                                     (((1,), (1,)), ((), ())),
                                     preferred_element_type=jnp.float32)


def _moe_mlp(x16, prowT, w1, w3, w2, tile_expert, n_real):
    def we_idx(i, te, nr):
        return jnp.minimum(te[i], te[jnp.minimum(nr[0] - 1, MAX_TILES - 1)])

    grid_spec = pltpu.PrefetchScalarGridSpec(
        num_scalar_prefetch=2,
        grid=(MAX_TILES,),
        in_specs=[
            pl.BlockSpec((S, H), lambda i, te, nr: (0, 0)),
            pl.BlockSpec((S, TOPK), lambda i, te, nr: (0, 0)),
            pl.BlockSpec((1, F, H), lambda i, te, nr: (we_idx(i, te, nr), 0, 0)),
            pl.BlockSpec((1, F, H), lambda i, te, nr: (we_idx(i, te, nr), 0, 0)),
            pl.BlockSpec((1, H, F), lambda i, te, nr: (we_idx(i, te, nr), 0, 0)),
        ],
        out_specs=pl.BlockSpec((TROW, H), lambda i, te, nr: (i, 0)),
    )
    return pl.pallas_call(
        _mlp_body,
        grid_spec=grid_spec,
        out_shape=jax.ShapeDtypeStruct((MAX_ROWS, H), jnp.float32),
        compiler_params=pltpu.CompilerParams(
            dimension_semantics=("arbitrary",)),
    )(tile_expert, n_real, x16, prowT, w1, w3, w2)


_TPW = S // _NW


def _sc_cparams():
    cp = pltpu.CompilerParams()
    if "needs_layout_passes" in pltpu.CompilerParams.__dataclass_fields__:
        cp = dataclasses.replace(cp, needs_layout_passes=False)
    return cp


def _sc_combine(y, pos0, pos1, w_flat):
    @functools.partial(
        pl.kernel,
        out_type=jax.ShapeDtypeStruct((S, H), jnp.float32),
        mesh=plsc.VectorSubcoreMesh(core_axis_name="c", subcore_axis_name="s"),
        compiler_params=_sc_cparams(),
        scratch_types=[
            pltpu.VMEM((_TPW,), jnp.int32),
            pltpu.VMEM((_TPW,), jnp.int32),
            pltpu.VMEM((2 * _TPW,), jnp.float32),
            pltpu.VMEM((_TPW, H), jnp.float32),
            pltpu.VMEM((_TPW, H), jnp.float32),
            pltpu.SemaphoreType.DMA,
        ],
    )
    def k(y_hbm, p0_hbm, p1_hbm, w_hbm, out_hbm, i0, i1, wv, b0, b1, sem):
        wid = lax.axis_index("s") * _NC + lax.axis_index("c")
        base = wid * _TPW
        pltpu.sync_copy(p0_hbm.at[pl.ds(base, _TPW)], i0)
        pltpu.sync_copy(p1_hbm.at[pl.ds(base, _TPW)], i1)
        pltpu.sync_copy(w_hbm.at[pl.ds(2 * base, 2 * _TPW)], wv)
        c0 = pltpu.async_copy(y_hbm.at[i0], b0, sem)
        c1 = pltpu.async_copy(y_hbm.at[i1], b1, sem)
        c0.wait()
        c1.wait()

        @pl.loop(0, _TPW)
        def _(r):
            w0 = plsc.load_gather(wv, [jnp.full((16,), 2 * r, jnp.int32)])
            w1 = plsc.load_gather(wv, [jnp.full((16,), 2 * r + 1, jnp.int32)])

            @pl.loop(0, H, step=16)
            def _(c):
                b0[r, pl.ds(c, 16)] = (b0[r, pl.ds(c, 16)] * w0 +
                                       b1[r, pl.ds(c, 16)] * w1)

        pltpu.sync_copy(b0, out_hbm.at[pl.ds(base, _TPW)])

    return k(y, pos0, pos1, w_flat)


def kernel(hidden_states, Wg, W1, W3, W2):
    x = hidden_states.reshape(S, H)
    logits, wts, prow2, tile_expert, n_real = _router(x, Wg)

    y = _moe_mlp(x.astype(jnp.bfloat16), prow2, W1, W3, W2,
                 tile_expert.reshape(MAX_TILES), n_real.reshape(1))
    final = _sc_combine(y, prow2[:, 0], prow2[:, 1], wts.reshape(A))
    return final.reshape(hidden_states.shape), logits


# speedup vs baseline: 1.0518x; 1.0306x over previous
import dataclasses
import functools

import jax
import jax.numpy as jnp
from jax import lax
from jax.experimental import pallas as pl
from jax.experimental.pallas import tpu as pltpu
from jax.experimental.pallas import tpu_sc as plsc

S = 2048
H = 768
F = 2048
E = 64
TOPK = 2
A = S * TOPK
TROW = 128
MAX_TILES = 96
MAX_ROWS = MAX_TILES * TROW

_NC = 2
_NS = 16
_NW = _NC * _NS
_SBLK = 256


_NBLK = S // _SBLK


def _router_body(x_ref, wg_ref, lg_ref, wt_ref, p0_ref, p1_ref, te_ref,
                 nr_ref, cnt_ref, sel_s, rank_s):
    i = pl.program_id(0)

    @pl.when(i == 0)
    def _():
        cnt_ref[...] = jnp.zeros((1, E), jnp.float32)

    @pl.when(i < _NBLK)
    def _():
        lg = lax.dot_general(x_ref[...], wg_ref[...], (((1,), (1,)), ((), ())),
                             preferred_element_type=jnp.float32)
        lg_ref[...] = lg
        iota = lax.broadcasted_iota(jnp.int32, lg.shape, 1)
        m1 = jnp.max(lg, axis=1, keepdims=True)
        e1 = jnp.min(jnp.where(lg == m1, iota, E), axis=1, keepdims=True)
        lg2 = jnp.where(iota == e1, -jnp.inf, lg)
        m2 = jnp.max(lg2, axis=1, keepdims=True)
        e2 = jnp.min(jnp.where(lg2 == m2, iota, E), axis=1, keepdims=True)
        w1 = 1.0 / (1.0 + jnp.exp(m2 - m1))
        wt_ref[...] = jnp.concatenate([w1, 1.0 - w1], axis=1)
        sel_s[pl.ds(i * _SBLK, _SBLK), :] = jnp.concatenate([e1, e2], axis=1)

        oh0 = (iota == e1).astype(jnp.bfloat16)
        oh1 = (iota == e2).astype(jnp.bfloat16)
        rr = lax.broadcasted_iota(jnp.int32, (_SBLK, _SBLK), 0)
        cc = lax.broadcasted_iota(jnp.int32, (_SBLK, _SBLK), 1)
        ltri = (rr > cc).astype(jnp.bfloat16)
        p0 = lax.dot_general(ltri, oh0, (((1,), (0,)), ((), ())),
                             preferred_element_type=jnp.float32)
        p1 = lax.dot_general(ltri, oh1, (((1,), (0,)), ((), ())),
                             preferred_element_type=jnp.float32)
        oh0f = oh0.astype(jnp.float32)
        oh1f = oh1.astype(jnp.float32)
        s0 = jnp.sum(oh0f, axis=0, keepdims=True)
        cnt = cnt_ref[...]
        rank0 = jnp.sum(oh0f * (p0 + cnt), axis=1, keepdims=True)
        rank1 = jnp.sum(oh1f * (p1 + cnt + s0), axis=1, keepdims=True)
        rank_s[pl.ds(i * _SBLK, _SBLK), :] = jnp.concatenate(
            [rank0, rank1], axis=1)
        cnt_ref[...] = cnt + s0 + jnp.sum(oh1f, axis=0, keepdims=True)

    @pl.when(i == _NBLK)
    def _():
        cnt = cnt_ref[...]
        n_tiles = jnp.floor((cnt + (TROW - 1)) * (1.0 / TROW))
        iot = lax.broadcasted_iota(jnp.int32, (E, E), 0)
        iot2 = lax.broadcasted_iota(jnp.int32, (E, E), 1)
        ut = (iot <= iot2).astype(jnp.bfloat16)
        cum_tiles = lax.dot_general(n_tiles.astype(jnp.bfloat16), ut,
                                    (((1,), (0,)), ((), ())),
                                    preferred_element_type=jnp.float32)
        row_base = (cum_tiles - n_tiles) * TROW
        nr_ref[...] = cum_tiles[:, E - 1:].astype(jnp.int32)
        tt = lax.broadcasted_iota(jnp.int32, (MAX_TILES, E), 0)
        cum_i = cum_tiles.astype(jnp.int32)
        te = jnp.sum((jnp.broadcast_to(cum_i, (MAX_TILES, E)) <= tt)
                     .astype(jnp.int32), axis=1, keepdims=True)
        te_ref[...] = jnp.minimum(te, E - 1)
        sel = sel_s[...]
        iota_e0 = lax.broadcasted_iota(jnp.int32, (S, E), 1)
        rb = jnp.broadcast_to(row_base, (S, E))
        rb0 = jnp.sum(jnp.where(iota_e0 == sel[:, 0:1], rb, 0.0),
                      axis=1, keepdims=True)
        rb1 = jnp.sum(jnp.where(iota_e0 == sel[:, 1:2], rb, 0.0),
                      axis=1, keepdims=True)
        p0_ref[...] = (rb0 + rank_s[:, 0:1]).astype(jnp.int32)
        p1_ref[...] = (rb1 + rank_s[:, 1:2]).astype(jnp.int32)


def _router(x, wg):
    return pl.pallas_call(
        _router_body,
        grid=(_NBLK + 1,),
        in_specs=[
            pl.BlockSpec((_SBLK, H), lambda i: (jnp.minimum(i, _NBLK - 1), 0)),
            pl.BlockSpec((E, H), lambda i: (0, 0)),
        ],
        out_specs=[
            pl.BlockSpec((_SBLK, E), lambda i: (jnp.minimum(i, _NBLK - 1), 0)),
            pl.BlockSpec((_SBLK, TOPK),
                         lambda i: (jnp.minimum(i, _NBLK - 1), 0)),
            pl.BlockSpec((S, 1), lambda i: (0, 0)),
            pl.BlockSpec((S, 1), lambda i: (0, 0)),
            pl.BlockSpec((MAX_TILES, 1), lambda i: (0, 0)),
            pl.BlockSpec((1, 1), lambda i: (0, 0)),
        ],
        out_shape=[
            jax.ShapeDtypeStruct((S, E), jnp.float32),
            jax.ShapeDtypeStruct((S, TOPK), jnp.float32),
            jax.ShapeDtypeStruct((S, 1), jnp.int32),
            jax.ShapeDtypeStruct((S, 1), jnp.int32),
            jax.ShapeDtypeStruct((MAX_TILES, 1), jnp.int32),
            jax.ShapeDtypeStruct((1, 1), jnp.int32),
        ],
        scratch_shapes=[
            pltpu.VMEM((1, E), jnp.float32),
            pltpu.VMEM((S, TOPK), jnp.int32),
            pltpu.VMEM((S, TOPK), jnp.float32),
        ],
    )(x, wg)


def _mlp_body(te_ref, nr_ref, x_ref, p0_ref, p1_ref, w1_ref, w3_ref, w2_ref,
              y_ref):
    i = pl.program_id(0)

    @pl.when(i < nr_ref[0])
    def _():
        rowidx = i * TROW + lax.broadcasted_iota(jnp.int32, (S, TROW), 1)
        oh = ((p0_ref[...] == rowidx) |
              (p1_ref[...] == rowidx)).astype(jnp.bfloat16)
        xb = lax.dot_general(oh, x_ref[...], (((0,), (0,)), ((), ())),
                             preferred_element_type=jnp.float32
                             ).astype(jnp.bfloat16)
        a = lax.dot_general(xb, w1_ref[0].astype(jnp.bfloat16),
                            (((1,), (1,)), ((), ())),
                            preferred_element_type=jnp.float32)
        b = lax.dot_general(xb, w3_ref[0].astype(jnp.bfloat16),
                            (((1,), (1,)), ((), ())),
                            preferred_element_type=jnp.float32)
        h = (a * jax.nn.sigmoid(a)) * b
        y_ref[...] = lax.dot_general(h.astype(jnp.bfloat16),
                                     w2_ref[0].astype(jnp.bfloat16),
                                     (((1,), (1,)), ((), ())),
                                     preferred_element_type=jnp.float32)


def _moe_mlp(x16, p0, p1, w1, w3, w2, tile_expert, n_real):
    def we_idx(i, te, nr):
        return jnp.minimum(te[i], te[jnp.minimum(nr[0] - 1, MAX_TILES - 1)])

    grid_spec = pltpu.PrefetchScalarGridSpec(
        num_scalar_prefetch=2,
        grid=(MAX_TILES,),
        in_specs=[
            pl.BlockSpec((S, H), lambda i, te, nr: (0, 0)),
            pl.BlockSpec((S, 1), lambda i, te, nr: (0, 0)),
            pl.BlockSpec((S, 1), lambda i, te, nr: (0, 0)),
            pl.BlockSpec((1, F, H), lambda i, te, nr: (we_idx(i, te, nr), 0, 0)),
            pl.BlockSpec((1, F, H), lambda i, te, nr: (we_idx(i, te, nr), 0, 0)),
            pl.BlockSpec((1, H, F), lambda i, te, nr: (we_idx(i, te, nr), 0, 0)),
        ],
        out_specs=pl.BlockSpec(
            (TROW, H),
            lambda i, te, nr: (jnp.minimum(i, nr[0] - 1), 0)),
    )
    return pl.pallas_call(
        _mlp_body,
        grid_spec=grid_spec,
        out_shape=jax.ShapeDtypeStruct((MAX_ROWS, H), jnp.float32),
        compiler_params=pltpu.CompilerParams(
            dimension_semantics=("arbitrary",)),
    )(tile_expert, n_real, x16, p0, p1, w1, w3, w2)


_TPW = S // _NW


def _sc_cparams():
    cp = pltpu.CompilerParams()
    if "needs_layout_passes" in pltpu.CompilerParams.__dataclass_fields__:
        cp = dataclasses.replace(cp, needs_layout_passes=False)
    return cp


def _sc_combine(y, pos0, pos1, w_flat):
    @functools.partial(
        pl.kernel,
        out_type=jax.ShapeDtypeStruct((S, H), jnp.float32),
        mesh=plsc.VectorSubcoreMesh(core_axis_name="c", subcore_axis_name="s"),
        compiler_params=_sc_cparams(),
        scratch_types=[
            pltpu.VMEM((_TPW,), jnp.int32),
            pltpu.VMEM((_TPW,), jnp.int32),
            pltpu.VMEM((2 * _TPW,), jnp.float32),
            pltpu.VMEM((_TPW, H), jnp.float32),
            pltpu.VMEM((_TPW, H), jnp.float32),
            pltpu.SemaphoreType.DMA,
        ],
    )
    def k(y_hbm, p0_hbm, p1_hbm, w_hbm, out_hbm, i0, i1, wv, b0, b1, sem):
        wid = lax.axis_index("s") * _NC + lax.axis_index("c")
        base = wid * _TPW
        pltpu.sync_copy(p0_hbm.at[pl.ds(base, _TPW)], i0)
        pltpu.sync_copy(p1_hbm.at[pl.ds(base, _TPW)], i1)
        pltpu.sync_copy(w_hbm.at[pl.ds(2 * base, 2 * _TPW)], wv)
        c0 = pltpu.async_copy(y_hbm.at[i0], b0, sem)
        c1 = pltpu.async_copy(y_hbm.at[i1], b1, sem)
        c0.wait()
        c1.wait()

        @pl.loop(0, _TPW)
        def _(r):
            w0 = plsc.load_gather(wv, [jnp.full((16,), 2 * r, jnp.int32)])
            w1 = plsc.load_gather(wv, [jnp.full((16,), 2 * r + 1, jnp.int32)])

            @pl.loop(0, H, step=16)
            def _(c):
                b0[r, pl.ds(c, 16)] = (b0[r, pl.ds(c, 16)] * w0 +
                                       b1[r, pl.ds(c, 16)] * w1)

        pltpu.sync_copy(b0, out_hbm.at[pl.ds(base, _TPW)])

    return k(y, pos0, pos1, w_flat)


def kernel(hidden_states, Wg, W1, W3, W2):
    x = hidden_states.reshape(S, H)
    logits, wts, p0, p1, tile_expert, n_real = _router(x, Wg)

    y = _moe_mlp(x.astype(jnp.bfloat16), p0, p1, W1, W3, W2,
                 tile_expert.reshape(MAX_TILES), n_real.reshape(1))
    final = _sc_combine(y, p0.reshape(S), p1.reshape(S), wts.reshape(A))
    return final.reshape(hidden_states.shape), logits
